# baseline (speedup 1.0000x reference)
import functools

import jax
import jax.numpy as jnp
from jax import lax
from jax.experimental import pallas as pl
from jax.experimental.pallas import tpu as pltpu
from jax.experimental.pallas import tpu_sc as plsc

N = 10000
E = 320000
C = 128
CH = C // 4
D_EDGE = 16

NC = 2
NS = 16
NW = NC * NS
EW = E // NW
RW = 80
NR = EW // RW
NPAD = 10240
ZR = NPAD // NS
NB = 5

_f32 = jnp.float32
_i32 = jnp.int32

_SC_PARAMS = pltpu.CompilerParams(
    needs_layout_passes=False, use_tc_tiling_on_sc=False)


def _tc_nodes_body(x_ref, wq_ref, wk_ref, wv_ref, nb_ref, w1_ref, w2_ref,
                   v0_ref, v1_ref, v2_ref, v3_ref, ad_ref, as_ref):
    xv = x_ref[...]
    nb = nb_ref[...]
    q = jnp.dot(xv, wq_ref[...], preferred_element_type=_f32)
    k = jnp.dot(xv, wk_ref[...], preferred_element_type=_f32)
    v = jnp.dot(xv, wv_ref[...], preferred_element_type=_f32)
    v0_ref[...] = v[:, 0 * CH:1 * CH]
    v1_ref[...] = v[:, 1 * CH:2 * CH]
    v2_ref[...] = v[:, 2 * CH:3 * CH]
    v3_ref[...] = v[:, 3 * CH:4 * CH]
    ad_ref[...] = jnp.dot(jnp.tanh(q + nb), w1_ref[...], preferred_element_type=_f32)
    as_ref[...] = jnp.dot(jnp.tanh(k + nb), w2_ref[...], preferred_element_type=_f32)


def _tc_edges_body(ea_ref, we_ref, w4_ref, rel_ref, w3_ref, out_ref):
    eb = jnp.dot(ea_ref[...], we_ref[...], preferred_element_type=_f32)
    a4 = jnp.dot(jnp.tanh(eb), w4_ref[...], preferred_element_type=_f32)
    crel = jnp.dot(jnp.tanh(rel_ref[...]), w3_ref[...], preferred_element_type=_f32)
    out_ref[...] = a4 + crel[0, 0]


def _tc_final_body(a00_ref, a10_ref, a01_ref, a11_ref, a02_ref, a12_ref,
                   a03_ref, a13_ref, t0_ref, t1_ref,
                   den0_ref, den1_ref, x_ref, rel_ref, we_ref, wout_ref,
                   wroot_ref, b_ref, out_ref):
    d = den0_ref[...] + den1_ref[...]
    inv = 1.0 / (d + 1e-16)
    s = d * inv
    agg_q = [(a00_ref[...] + a10_ref[...]) * inv,
             (a01_ref[...] + a11_ref[...]) * inv,
             (a02_ref[...] + a12_ref[...]) * inv,
             (a03_ref[...] + a13_ref[...]) * inv]
    agg = (jnp.concatenate(agg_q, axis=1)
           + s * rel_ref[...]
           + jnp.dot((t0_ref[...] + t1_ref[...]) * inv, we_ref[...],
                     preferred_element_type=_f32))
    out_ref[...] = (jnp.dot(agg, wout_ref[...], preferred_element_type=_f32)
                    + jnp.dot(x_ref[...], wroot_ref[...], preferred_element_type=_f32)
                    + b_ref[...])


@functools.cache
def _sc_aggregate_kernel():
    mesh = plsc.VectorSubcoreMesh(core_axis_name="c", subcore_axis_name="s")

    @functools.partial(
        pl.kernel,
        out_type=[jax.ShapeDtypeStruct((N, CH), _f32),
                  jax.ShapeDtypeStruct((N, CH), _f32),
                  jax.ShapeDtypeStruct((N, CH), _f32),
                  jax.ShapeDtypeStruct((N, CH), _f32),
                  jax.ShapeDtypeStruct((N, CH), _f32),
                  jax.ShapeDtypeStruct((N, CH), _f32),
                  jax.ShapeDtypeStruct((N, CH), _f32),
                  jax.ShapeDtypeStruct((N, CH), _f32),
                  jax.ShapeDtypeStruct((N, D_EDGE), _f32),
                  jax.ShapeDtypeStruct((N, D_EDGE), _f32),
                  jax.ShapeDtypeStruct((N,), _f32),
                  jax.ShapeDtypeStruct((N,), _f32)],
        mesh=mesh,
        compiler_params=_SC_PARAMS,
        scratch_types=[
            pltpu.VMEM((N,), _f32),
            pltpu.VMEM((N,), _f32),
            pltpu.VMEM((NR, RW), _i32),
            pltpu.VMEM((NR, RW), _i32),
            pltpu.VMEM((NR, RW), _f32),
            pltpu.VMEM((2 * NB, RW, CH), _f32),
            pltpu.VMEM((NB, RW, D_EDGE), _f32),
            pltpu.VMEM((RW, CH), _f32),
            pltpu.VMEM((RW, D_EDGE), _f32),
            pltpu.VMEM((ZR,), _f32),
            pltpu.VMEM_SHARED((NPAD,), _f32),
            pltpu.VMEM_SHARED((N, CH), _f32),
            pltpu.VMEM_SHARED((N, D_EDGE), _f32),
            pltpu.SemaphoreType.DMA,
            pltpu.SemaphoreType.DMA,
            pltpu.SemaphoreType.DMA,
        ],
    )
    def sc_aggregate(col_hbm, row_hbm, ea_hbm, ad_hbm, as_hbm,
                     v0_hbm, v1_hbm, v2_hbm, v3_hbm, eattr_hbm,
                     a00_out, a10_out, a01_out, a11_out, a02_out, a12_out,
                     a03_out, a13_out, t0_out, t1_out,
                     den0_out, den1_out,
                     adbuf, asbuf, colbuf, rowbuf, eabuf, vrows, earows,
                     zbufv, zbuf2, zbuf1, denom_sh, aggv_sh, t_sh,
                     gsem, ssem, tsem):
        c = lax.axis_index("c")
        s = lax.axis_index("s")
        wid = s * NC + c
        ebase = wid * EW

        z16 = jnp.zeros((16,), _f32)

        def zfillv(i, carry):
            zbufv[i // (CH // 16), pl.ds((i % (CH // 16)) * 16, 16)] = z16
            return carry

        lax.fori_loop(0, RW * (CH // 16), zfillv, 0)

        def zfill2(i, carry):
            zbuf2[i, pl.ds(0, 16)] = z16
            return carry

        lax.fori_loop(0, RW, zfill2, 0)

        def zfill1(i, carry):
            zbuf1[pl.ds(i * 16, 16)] = z16
            return carry

        lax.fori_loop(0, ZR // 16, zfill1, 0)

        nchunk_z = jnp.where(s < NS - 1, ZR // RW, (N - (NS - 1) * ZR) // RW)

        def zero_aggv():
            def zc(j, carry):
                zoff = pl.multiple_of(s * ZR + j * RW, RW)
                pltpu.sync_copy(zbufv, aggv_sh.at[pl.ds(zoff, RW)])
                return carry

            lax.fori_loop(0, nchunk_z, zc, 0)

        zero_aggv()

        def ztc(j, carry):
            zoff = pl.multiple_of(s * ZR + j * RW, RW)
            pltpu.sync_copy(zbuf2, t_sh.at[pl.ds(zoff, RW)])
            return carry

        lax.fori_loop(0, nchunk_z, ztc, 0)
        pltpu.sync_copy(zbuf1, denom_sh.at[pl.ds(pl.multiple_of(s * ZR, ZR), ZR)])

        pltpu.sync_copy(ad_hbm, adbuf)
        pltpu.sync_copy(as_hbm, asbuf)
        pltpu.sync_copy(col_hbm.at[wid], colbuf)
        pltpu.sync_copy(row_hbm.at[wid], rowbuf)
        pltpu.sync_copy(ea_hbm.at[wid], eabuf)
        plsc.subcore_barrier()

        NWIN = NR // NB

        def edge_pass(v_hbm, first):
            def fire(r, boff):
                pltpu.async_copy(v_hbm.at[rowbuf.at[r]], vrows.at[boff], gsem)

            def drain_scatters(boff):
                for b in range(NB):
                    pltpu.make_async_copy(
                        vrows.at[boff + b], aggv_sh.at[colbuf.at[0]],
                        ssem).wait()

            for b in range(NB):
                fire(b, b)

            def window(g, carry):
                poff = lax.rem(g, 2) * NB
                noff = NB - poff

                @pl.when(g + 1 < NWIN)
                def _prefetch():
                    @pl.when(g > 0)
                    def _drain_prev():
                        drain_scatters(noff)

                    for b in range(NB):
                        fire((g + 1) * NB + b, noff + b)

                for b in range(NB):
                    r = g * NB + b
                    rv16 = jnp.full((16,), r, _i32)
                    if first:
                        def xbody(kk, kcarry):
                            sl = pl.ds(kk * 16, 16)
                            adv = plsc.load_gather(adbuf, [colbuf[r, sl]])
                            asv = plsc.load_gather(asbuf, [rowbuf[r, sl]])
                            eabuf[r, sl] = jnp.exp(adv + asv + eabuf[r, sl])
                            return kcarry

                        lax.fori_loop(0, RW // 16, xbody, 0)
                    pltpu.make_async_copy(
                        v_hbm.at[rowbuf.at[r]], vrows.at[poff + b], gsem).wait()
                    if first:
                        @pl.when(g > 0)
                        def _drain_t(b=b):
                            pltpu.make_async_copy(
                                earows.at[b], t_sh.at[colbuf.at[0]],
                                tsem).wait()

                        eoff = pl.multiple_of(ebase + r * RW, RW)
                        pltpu.sync_copy(
                            eattr_hbm.at[pl.ds(eoff, RW)], earows.at[b])

                    def kbody(kk, kcarry, rv16=rv16, poff=poff, b=b):
                        for j in range(16):
                            e = kk * 16 + j
                            ev16 = jnp.full((16,), e, _i32)
                            bc = plsc.load_gather(eabuf, [rv16, ev16])
                            for f in range(CH // 16):
                                fs = pl.ds(f * 16, 16)
                                vrows[poff + b, e, fs] = vrows[poff + b, e, fs] * bc
                            if first:
                                earows[b, e, pl.ds(0, 16)] = (
                                    earows[b, e, pl.ds(0, 16)] * bc)
                        return kcarry

                    lax.fori_loop(0, RW // 16, kbody, 0)
                    pltpu.async_copy(
                        vrows.at[poff + b], aggv_sh.at[colbuf.at[r]], ssem,
                        add=True)
                    if first:
                        pltpu.async_copy(
                            earows.at[b], t_sh.at[colbuf.at[r]], tsem,
                            add=True)
                        pltpu.sync_copy(eabuf.at[r],
                                        denom_sh.at[colbuf.at[r]], add=True)
                return carry

            lax.fori_loop(0, NWIN, window, 0)
            drain_scatters(((NWIN - 2) % 2) * NB)
            drain_scatters(((NWIN - 1) % 2) * NB)
            if first:
                for b in range(NB):
                    pltpu.make_async_copy(
                        earows.at[b], t_sh.at[colbuf.at[0]], tsem).wait()

        def dump_aggv(out0, out1):
            def dump_chunk(j, out_ref):
                doff = pl.multiple_of(s * ZR + j * RW, RW)
                pltpu.sync_copy(aggv_sh.at[pl.ds(doff, RW)], vrows.at[0])
                pltpu.sync_copy(vrows.at[0], out_ref.at[pl.ds(doff, RW)])
                return 0

            nchunk = jnp.where(s < NS - 1, ZR // RW, (N - (NS - 1) * ZR) // RW)

            @pl.when(c == 0)
            def _d0():
                lax.fori_loop(0, nchunk, lambda j, cy: dump_chunk(j, out0), 0)

            @pl.when(c == 1)
            def _d1():
                lax.fori_loop(0, nchunk, lambda j, cy: dump_chunk(j, out1), 0)

        edge_pass(v0_hbm, True)
        plsc.subcore_barrier()
        dump_aggv(a00_out, a10_out)

        def dump_t(j, out_ref):
            doff = pl.multiple_of(s * ZR + j * RW, RW)
            pltpu.sync_copy(t_sh.at[pl.ds(doff, RW)], earows.at[0])
            pltpu.sync_copy(earows.at[0], out_ref.at[pl.ds(doff, RW)])
            return 0

        nchunk_t = jnp.where(s < NS - 1, ZR // RW, (N - (NS - 1) * ZR) // RW)

        @pl.when(c == 0)
        def _dt0():
            lax.fori_loop(0, nchunk_t, lambda j, cy: dump_t(j, t0_out), 0)

        @pl.when(c == 1)
        def _dt1():
            lax.fori_loop(0, nchunk_t, lambda j, cy: dump_t(j, t1_out), 0)

        @pl.when((s == 0) & (c == 0))
        def _dden0():
            pltpu.sync_copy(denom_sh.at[pl.ds(0, N)], adbuf)
            pltpu.sync_copy(adbuf, den0_out)

        @pl.when((s == 0) & (c == 1))
        def _dden1():
            pltpu.sync_copy(denom_sh.at[pl.ds(0, N)], adbuf)
            pltpu.sync_copy(adbuf, den1_out)

        for v_hbm, o0, o1 in ((v1_hbm, a01_out, a11_out),
                              (v2_hbm, a02_out, a12_out),
                              (v3_hbm, a03_out, a13_out)):
            zero_aggv()
            plsc.subcore_barrier()
            edge_pass(v_hbm, False)
            plsc.subcore_barrier()
            dump_aggv(o0, o1)

    return sc_aggregate


def kernel(x, edge_index, edge_attr, Wq, Wk, Wv, node_emb, rel_emb, W_edge,
           W_attn, W_out, b_out, W_root):
    row3 = edge_index[0].reshape(NW, NR, RW)
    col3 = edge_index[1].reshape(NW, NR, RW)
    w1 = W_attn[:C]
    w2 = W_attn[C:2 * C]
    w3 = W_attn[2 * C:3 * C]
    w4 = W_attn[3 * C:]
    nb = node_emb.reshape(1, C)
    rel = rel_emb.reshape(1, C)
    b2 = b_out.reshape(1, C)

    grid_n = 10
    bn = N // grid_n
    v0, v1, v2, v3, ad, as_ = pl.pallas_call(
        _tc_nodes_body,
        grid=(grid_n,),
        in_specs=[
            pl.BlockSpec((bn, C), lambda i: (i, 0)),
            pl.BlockSpec((C, C), lambda i: (0, 0)),
            pl.BlockSpec((C, C), lambda i: (0, 0)),
            pl.BlockSpec((C, C), lambda i: (0, 0)),
            pl.BlockSpec((1, C), lambda i: (0, 0)),
            pl.BlockSpec((C, 1), lambda i: (0, 0)),
            pl.BlockSpec((C, 1), lambda i: (0, 0)),
        ],
        out_specs=[
            pl.BlockSpec((bn, CH), lambda i: (i, 0)),
            pl.BlockSpec((bn, CH), lambda i: (i, 0)),
            pl.BlockSpec((bn, CH), lambda i: (i, 0)),
            pl.BlockSpec((bn, CH), lambda i: (i, 0)),
            pl.BlockSpec((bn, 1), lambda i: (i, 0)),
            pl.BlockSpec((bn, 1), lambda i: (i, 0)),
        ],
        out_shape=[
            jax.ShapeDtypeStruct((N, CH), _f32),
            jax.ShapeDtypeStruct((N, CH), _f32),
            jax.ShapeDtypeStruct((N, CH), _f32),
            jax.ShapeDtypeStruct((N, CH), _f32),
            jax.ShapeDtypeStruct((N, 1), _f32),
            jax.ShapeDtypeStruct((N, 1), _f32),
        ],
    )(x, Wq, Wk, Wv, nb, w1, w2)

    grid_e = 100
    be = E // grid_e
    ea = pl.pallas_call(
        _tc_edges_body,
        grid=(grid_e,),
        in_specs=[
            pl.BlockSpec((be, D_EDGE), lambda i: (i, 0)),
            pl.BlockSpec((D_EDGE, C), lambda i: (0, 0)),
            pl.BlockSpec((C, 1), lambda i: (0, 0)),
            pl.BlockSpec((1, C), lambda i: (0, 0)),
            pl.BlockSpec((C, 1), lambda i: (0, 0)),
        ],
        out_specs=pl.BlockSpec((be, 1), lambda i: (i, 0)),
        out_shape=jax.ShapeDtypeStruct((E, 1), _f32),
    )(edge_attr, W_edge, w4, rel, w3)

    (a00, a10, a01, a11, a02, a12, a03, a13, t0, t1, den0,
     den1) = _sc_aggregate_kernel()(
        col3, row3, ea.reshape(NW, NR, RW), ad.reshape(N), as_.reshape(N),
        v0, v1, v2, v3, edge_attr)

    out = pl.pallas_call(
        _tc_final_body,
        grid=(grid_n,),
        in_specs=[
            pl.BlockSpec((bn, CH), lambda i: (i, 0)),
            pl.BlockSpec((bn, CH), lambda i: (i, 0)),
            pl.BlockSpec((bn, CH), lambda i: (i, 0)),
            pl.BlockSpec((bn, CH), lambda i: (i, 0)),
            pl.BlockSpec((bn, CH), lambda i: (i, 0)),
            pl.BlockSpec((bn, CH), lambda i: (i, 0)),
            pl.BlockSpec((bn, CH), lambda i: (i, 0)),
            pl.BlockSpec((bn, CH), lambda i: (i, 0)),
            pl.BlockSpec((bn, D_EDGE), lambda i: (i, 0)),
            pl.BlockSpec((bn, D_EDGE), lambda i: (i, 0)),
            pl.BlockSpec((bn, 1), lambda i: (i, 0)),
            pl.BlockSpec((bn, 1), lambda i: (i, 0)),
            pl.BlockSpec((bn, C), lambda i: (i, 0)),
            pl.BlockSpec((1, C), lambda i: (0, 0)),
            pl.BlockSpec((D_EDGE, C), lambda i: (0, 0)),
            pl.BlockSpec((C, C), lambda i: (0, 0)),
            pl.BlockSpec((C, C), lambda i: (0, 0)),
            pl.BlockSpec((1, C), lambda i: (0, 0)),
        ],
        out_specs=pl.BlockSpec((bn, C), lambda i: (i, 0)),
        out_shape=jax.ShapeDtypeStruct((N, C), _f32),
    )(a00, a10, a01, a11, a02, a12, a03, a13, t0, t1,
      den0.reshape(N, 1), den1.reshape(N, 1),
      x, rel, W_edge, W_out, W_root, b2)

    return out


# speedup vs baseline: 1.0001x; 1.0001x over previous
import functools

import jax
import jax.numpy as jnp
from jax import lax
from jax.experimental import pallas as pl
from jax.experimental.pallas import tpu as pltpu
from jax.experimental.pallas import tpu_sc as plsc

N = 10000
E = 320000
C = 128
CH = C // 2
D_EDGE = 16

NC = 2
NS = 16
NW = NC * NS
EW = E // NW
RW = 80
NR = EW // RW
NPAD = 10240
ZR = NPAD // NS
NB = 5

_f32 = jnp.float32
_i32 = jnp.int32

_SC_PARAMS = pltpu.CompilerParams(
    needs_layout_passes=False, use_tc_tiling_on_sc=False)


def _tc_nodes_body(x_ref, wq_ref, wk_ref, wv_ref, nb_ref, w1_ref, w2_ref,
                   va_ref, vb_ref, ad_ref, as_ref):
    xv = x_ref[...]
    nb = nb_ref[...]
    q = jnp.dot(xv, wq_ref[...], preferred_element_type=_f32)
    k = jnp.dot(xv, wk_ref[...], preferred_element_type=_f32)
    v = jnp.dot(xv, wv_ref[...], preferred_element_type=_f32)
    va_ref[...] = v[:, :CH]
    vb_ref[...] = v[:, CH:]
    ad_ref[...] = jnp.dot(jnp.tanh(q + nb), w1_ref[...], preferred_element_type=_f32)
    as_ref[...] = jnp.dot(jnp.tanh(k + nb), w2_ref[...], preferred_element_type=_f32)


def _tc_edges_body(ea_ref, we_ref, w4_ref, rel_ref, w3_ref, out_ref):
    eb = jnp.dot(ea_ref[...], we_ref[...], preferred_element_type=_f32)
    a4 = jnp.dot(jnp.tanh(eb), w4_ref[...], preferred_element_type=_f32)
    crel = jnp.dot(jnp.tanh(rel_ref[...]), w3_ref[...], preferred_element_type=_f32)
    out_ref[...] = a4 + crel[0, 0]


def _tc_final_body(a0a_ref, a1a_ref, a0b_ref, a1b_ref, t0_ref, t1_ref,
                   den0_ref, den1_ref, x_ref, rel_ref, we_ref, wout_ref,
                   wroot_ref, b_ref, out_ref):
    d = den0_ref[...] + den1_ref[...]
    inv = 1.0 / (d + 1e-16)
    s = d * inv
    agg_a = (a0a_ref[...] + a1a_ref[...]) * inv
    agg_b = (a0b_ref[...] + a1b_ref[...]) * inv
    agg = (jnp.concatenate([agg_a, agg_b], axis=1)
           + s * rel_ref[...]
           + jnp.dot((t0_ref[...] + t1_ref[...]) * inv, we_ref[...],
                     preferred_element_type=_f32))
    out_ref[...] = (jnp.dot(agg, wout_ref[...], preferred_element_type=_f32)
                    + jnp.dot(x_ref[...], wroot_ref[...], preferred_element_type=_f32)
                    + b_ref[...])


@functools.cache
def _sc_scores_kernel():
    mesh = plsc.VectorSubcoreMesh(core_axis_name="c", subcore_axis_name="s")

    @functools.partial(
        pl.kernel,
        out_type=[jax.ShapeDtypeStruct((NW, NR, RW), _f32),
                  jax.ShapeDtypeStruct((N,), _f32),
                  jax.ShapeDtypeStruct((N,), _f32)],
        mesh=mesh,
        compiler_params=_SC_PARAMS,
        scratch_types=[
            pltpu.VMEM((N,), _f32),
            pltpu.VMEM((N,), _f32),
            pltpu.VMEM((NR, RW), _i32),
            pltpu.VMEM((NR, RW), _i32),
            pltpu.VMEM((NR, RW), _f32),
            pltpu.VMEM((ZR,), _f32),
            pltpu.VMEM_SHARED((NPAD,), _f32),
            pltpu.SemaphoreType.DMA,
        ],
    )
    def sc_scores(col_hbm, row_hbm, ea_hbm, ad_hbm, as_hbm,
                  ex_out, den0_out, den1_out,
                  adbuf, asbuf, colbuf, rowbuf, eabuf, zbuf, denom_sh, dsem):
        c = lax.axis_index("c")
        s = lax.axis_index("s")
        wid = s * NC + c

        z16 = jnp.zeros((16,), _f32)

        def zfill(i, carry):
            zbuf[pl.ds(i * 16, 16)] = z16
            return carry

        lax.fori_loop(0, ZR // 16, zfill, 0)
        pltpu.sync_copy(zbuf, denom_sh.at[pl.ds(pl.multiple_of(s * ZR, ZR), ZR)])

        pltpu.sync_copy(ad_hbm, adbuf)
        pltpu.sync_copy(as_hbm, asbuf)
        pltpu.sync_copy(col_hbm.at[wid], colbuf)
        pltpu.sync_copy(row_hbm.at[wid], rowbuf)
        pltpu.sync_copy(ea_hbm.at[wid], eabuf)
        plsc.subcore_barrier()

        def body(r, carry):
            def kbody(kk, kcarry):
                sl = pl.ds(kk * 16, 16)
                adv = plsc.load_gather(adbuf, [colbuf[r, sl]])
                asv = plsc.load_gather(asbuf, [rowbuf[r, sl]])
                eabuf[r, sl] = jnp.exp(adv + asv + eabuf[r, sl])
                return kcarry

            lax.fori_loop(0, RW // 16, kbody, 0)
            pltpu.async_copy(eabuf.at[r], denom_sh.at[colbuf.at[r]], dsem,
                             add=True)
            return carry

        lax.fori_loop(0, NR, body, 0)

        def ddrain(r, carry):
            pltpu.make_async_copy(eabuf.at[0], denom_sh.at[colbuf.at[0]],
                                  dsem).wait()
            return carry

        lax.fori_loop(0, NR, ddrain, 0)
        pltpu.sync_copy(eabuf, ex_out.at[wid])
        plsc.subcore_barrier()

        @pl.when((s == 0) & (c == 0))
        def _dump0():
            pltpu.sync_copy(denom_sh.at[pl.ds(0, N)], adbuf)
            pltpu.sync_copy(adbuf, den0_out)

        @pl.when((s == 0) & (c == 1))
        def _dump1():
            pltpu.sync_copy(denom_sh.at[pl.ds(0, N)], adbuf)
            pltpu.sync_copy(adbuf, den1_out)

    return sc_scores


@functools.cache
def _sc_aggregate_kernel():
    mesh = plsc.VectorSubcoreMesh(core_axis_name="c", subcore_axis_name="s")

    @functools.partial(
        pl.kernel,
        out_type=[jax.ShapeDtypeStruct((N, CH), _f32),
                  jax.ShapeDtypeStruct((N, CH), _f32),
                  jax.ShapeDtypeStruct((N, CH), _f32),
                  jax.ShapeDtypeStruct((N, CH), _f32),
                  jax.ShapeDtypeStruct((N, D_EDGE), _f32),
                  jax.ShapeDtypeStruct((N, D_EDGE), _f32)],
        mesh=mesh,
        compiler_params=_SC_PARAMS,
        scratch_types=[
            pltpu.VMEM((NR, RW), _i32),
            pltpu.VMEM((NR, RW), _i32),
            pltpu.VMEM((NR, RW), _f32),
            pltpu.VMEM((NB, RW, CH), _f32),
            pltpu.VMEM((NB, RW, D_EDGE), _f32),
            pltpu.VMEM((RW, CH), _f32),
            pltpu.VMEM((RW, D_EDGE), _f32),
            pltpu.VMEM_SHARED((N, CH), _f32),
            pltpu.VMEM_SHARED((N, D_EDGE), _f32),
            pltpu.SemaphoreType.DMA,
            pltpu.SemaphoreType.DMA,
            pltpu.SemaphoreType.DMA,
        ],
    )
    def sc_aggregate(col_hbm, row_hbm, ex_hbm, va_hbm, vb_hbm, eattr_hbm,
                     a0a_out, a1a_out, a0b_out, a1b_out, t0_out, t1_out,
                     colbuf, rowbuf, exbuf, vrows, earows,
                     zbufv, zbuf2, aggv_sh, t_sh, gsem, ssem, tsem):
        c = lax.axis_index("c")
        s = lax.axis_index("s")
        wid = s * NC + c
        ebase = wid * EW

        z16 = jnp.zeros((16,), _f32)

        def zfillv(i, carry):
            zbufv[i // (CH // 16), pl.ds((i % (CH // 16)) * 16, 16)] = z16
            return carry

        lax.fori_loop(0, RW * (CH // 16), zfillv, 0)

        def zfill2(i, carry):
            zbuf2[i, pl.ds(0, 16)] = z16
            return carry

        lax.fori_loop(0, RW, zfill2, 0)

        nchunk_z = jnp.where(s < NS - 1, ZR // RW, (N - (NS - 1) * ZR) // RW)

        def zero_aggv():
            def zc(j, carry):
                zoff = pl.multiple_of(s * ZR + j * RW, RW)
                pltpu.sync_copy(zbufv, aggv_sh.at[pl.ds(zoff, RW)])
                return carry

            lax.fori_loop(0, nchunk_z, zc, 0)

        zero_aggv()

        def ztc(j, carry):
            zoff = pl.multiple_of(s * ZR + j * RW, RW)
            pltpu.sync_copy(zbuf2, t_sh.at[pl.ds(zoff, RW)])
            return carry

        lax.fori_loop(0, nchunk_z, ztc, 0)

        pltpu.sync_copy(col_hbm.at[wid], colbuf)
        pltpu.sync_copy(row_hbm.at[wid], rowbuf)
        pltpu.sync_copy(ex_hbm.at[wid], exbuf)
        plsc.subcore_barrier()

        NWIN = NR // NB

        def edge_pass(v_hbm, first):
            def fire(r, boff):
                pltpu.async_copy(v_hbm.at[rowbuf.at[r]], vrows.at[boff], gsem)

            def drain_scatters(boff):
                for b in range(NB):
                    pltpu.make_async_copy(
                        vrows.at[boff + b], aggv_sh.at[colbuf.at[0]],
                        ssem).wait()

            def window(g, carry):
                for b in range(NB):
                    fire(g * NB + b, b)
                for b in range(NB):
                    r = g * NB + b
                    rv16 = jnp.full((16,), r, _i32)
                    pltpu.make_async_copy(
                        v_hbm.at[rowbuf.at[r]], vrows.at[b], gsem).wait()
                    if first:
                        @pl.when(g > 0)
                        def _drain_t(b=b):
                            pltpu.make_async_copy(
                                earows.at[b], t_sh.at[colbuf.at[0]],
                                tsem).wait()

                        eoff = pl.multiple_of(ebase + r * RW, RW)
                        pltpu.sync_copy(
                            eattr_hbm.at[pl.ds(eoff, RW)], earows.at[b])

                    def kbody(kk, kcarry, rv16=rv16, b=b):
                        for j in range(16):
                            e = kk * 16 + j
                            ev16 = jnp.full((16,), e, _i32)
                            bc = plsc.load_gather(exbuf, [rv16, ev16])
                            for f in range(CH // 16):
                                fs = pl.ds(f * 16, 16)
                                vrows[b, e, fs] = vrows[b, e, fs] * bc
                            if first:
                                earows[b, e, pl.ds(0, 16)] = (
                                    earows[b, e, pl.ds(0, 16)] * bc)
                        return kcarry

                    lax.fori_loop(0, RW // 16, kbody, 0)
                    pltpu.async_copy(
                        vrows.at[b], aggv_sh.at[colbuf.at[r]], ssem,
                        add=True)
                    if first:
                        pltpu.async_copy(
                            earows.at[b], t_sh.at[colbuf.at[r]], tsem,
                            add=True)
                drain_scatters(0)
                return carry

            lax.fori_loop(0, NWIN, window, 0)
            if first:
                for b in range(NB):
                    pltpu.make_async_copy(
                        earows.at[b], t_sh.at[colbuf.at[0]], tsem).wait()

        def dump_aggv(out0, out1):
            def dump_chunk(j, out_ref):
                doff = pl.multiple_of(s * ZR + j * RW, RW)
                pltpu.sync_copy(aggv_sh.at[pl.ds(doff, RW)], vrows.at[0])
                pltpu.sync_copy(vrows.at[0], out_ref.at[pl.ds(doff, RW)])
                return 0

            @pl.when(c == 0)
            def _d0():
                lax.fori_loop(0, nchunk_z, lambda j, cy: dump_chunk(j, out0), 0)

            @pl.when(c == 1)
            def _d1():
                lax.fori_loop(0, nchunk_z, lambda j, cy: dump_chunk(j, out1), 0)

        edge_pass(va_hbm, True)
        plsc.subcore_barrier()
        dump_aggv(a0a_out, a1a_out)

        def dump_t(j, out_ref):
            doff = pl.multiple_of(s * ZR + j * RW, RW)
            pltpu.sync_copy(t_sh.at[pl.ds(doff, RW)], earows.at[0])
            pltpu.sync_copy(earows.at[0], out_ref.at[pl.ds(doff, RW)])
            return 0

        @pl.when(c == 0)
        def _dt0():
            lax.fori_loop(0, nchunk_z, lambda j, cy: dump_t(j, t0_out), 0)

        @pl.when(c == 1)
        def _dt1():
            lax.fori_loop(0, nchunk_z, lambda j, cy: dump_t(j, t1_out), 0)

        zero_aggv()
        plsc.subcore_barrier()

        edge_pass(vb_hbm, False)
        plsc.subcore_barrier()
        dump_aggv(a0b_out, a1b_out)

    return sc_aggregate


def kernel(x, edge_index, edge_attr, Wq, Wk, Wv, node_emb, rel_emb, W_edge,
           W_attn, W_out, b_out, W_root):
    row3 = edge_index[0].reshape(NW, NR, RW)
    col3 = edge_index[1].reshape(NW, NR, RW)
    w1 = W_attn[:C]
    w2 = W_attn[C:2 * C]
    w3 = W_attn[2 * C:3 * C]
    w4 = W_attn[3 * C:]
    nb = node_emb.reshape(1, C)
    rel = rel_emb.reshape(1, C)
    b2 = b_out.reshape(1, C)

    grid_n = 10
    bn = N // grid_n
    va, vb, ad, as_ = pl.pallas_call(
        _tc_nodes_body,
        grid=(grid_n,),
        in_specs=[
            pl.BlockSpec((bn, C), lambda i: (i, 0)),
            pl.BlockSpec((C, C), lambda i: (0, 0)),
            pl.BlockSpec((C, C), lambda i: (0, 0)),
            pl.BlockSpec((C, C), lambda i: (0, 0)),
            pl.BlockSpec((1, C), lambda i: (0, 0)),
            pl.BlockSpec((C, 1), lambda i: (0, 0)),
            pl.BlockSpec((C, 1), lambda i: (0, 0)),
        ],
        out_specs=[
            pl.BlockSpec((bn, CH), lambda i: (i, 0)),
            pl.BlockSpec((bn, CH), lambda i: (i, 0)),
            pl.BlockSpec((bn, 1), lambda i: (i, 0)),
            pl.BlockSpec((bn, 1), lambda i: (i, 0)),
        ],
        out_shape=[
            jax.ShapeDtypeStruct((N, CH), _f32),
            jax.ShapeDtypeStruct((N, CH), _f32),
            jax.ShapeDtypeStruct((N, 1), _f32),
            jax.ShapeDtypeStruct((N, 1), _f32),
        ],
    )(x, Wq, Wk, Wv, nb, w1, w2)

    grid_e = 100
    be = E // grid_e
    ea = pl.pallas_call(
        _tc_edges_body,
        grid=(grid_e,),
        in_specs=[
            pl.BlockSpec((be, D_EDGE), lambda i: (i, 0)),
            pl.BlockSpec((D_EDGE, C), lambda i: (0, 0)),
            pl.BlockSpec((C, 1), lambda i: (0, 0)),
            pl.BlockSpec((1, C), lambda i: (0, 0)),
            pl.BlockSpec((C, 1), lambda i: (0, 0)),
        ],
        out_specs=pl.BlockSpec((be, 1), lambda i: (i, 0)),
        out_shape=jax.ShapeDtypeStruct((E, 1), _f32),
    )(edge_attr, W_edge, w4, rel, w3)

    ex3, den0, den1 = _sc_scores_kernel()(col3, row3, ea.reshape(NW, NR, RW),
                                          ad.reshape(N), as_.reshape(N))

    a0a, a1a, a0b, a1b, t0, t1 = _sc_aggregate_kernel()(
        col3, row3, ex3, va, vb, edge_attr)

    out = pl.pallas_call(
        _tc_final_body,
        grid=(grid_n,),
        in_specs=[
            pl.BlockSpec((bn, CH), lambda i: (i, 0)),
            pl.BlockSpec((bn, CH), lambda i: (i, 0)),
            pl.BlockSpec((bn, CH), lambda i: (i, 0)),
            pl.BlockSpec((bn, CH), lambda i: (i, 0)),
            pl.BlockSpec((bn, D_EDGE), lambda i: (i, 0)),
            pl.BlockSpec((bn, D_EDGE), lambda i: (i, 0)),
            pl.BlockSpec((bn, 1), lambda i: (i, 0)),
            pl.BlockSpec((bn, 1), lambda i: (i, 0)),
            pl.BlockSpec((bn, C), lambda i: (i, 0)),
            pl.BlockSpec((1, C), lambda i: (0, 0)),
            pl.BlockSpec((D_EDGE, C), lambda i: (0, 0)),
            pl.BlockSpec((C, C), lambda i: (0, 0)),
            pl.BlockSpec((C, C), lambda i: (0, 0)),
            pl.BlockSpec((1, C), lambda i: (0, 0)),
        ],
        out_specs=pl.BlockSpec((bn, C), lambda i: (i, 0)),
        out_shape=jax.ShapeDtypeStruct((N, C), _f32),
    )(a0a, a1a, a0b, a1b, t0, t1, den0.reshape(N, 1), den1.reshape(N, 1),
      x, rel, W_edge, W_out, W_root, b2)

    return out


# speedup vs baseline: 1.0015x; 1.0014x over previous
import functools

import jax
import jax.numpy as jnp
from jax import lax
from jax.experimental import pallas as pl
from jax.experimental.pallas import tpu as pltpu
from jax.experimental.pallas import tpu_sc as plsc

N = 10000
E = 320000
C = 128
CH = C // 2
D_EDGE = 16

NC = 2
NS = 16
NW = NC * NS
EW = E // NW
RW = 80
NR = EW // RW
NPAD = 10240
ZR = NPAD // NS
NB = 5

_f32 = jnp.float32
_i32 = jnp.int32

_SC_PARAMS = pltpu.CompilerParams(
    needs_layout_passes=False, use_tc_tiling_on_sc=False)


def _tc_nodes_body(x_ref, wq_ref, wk_ref, wv_ref, nb_ref, w1_ref, w2_ref,
                   va_ref, vb_ref, ad_ref, as_ref):
    xv = x_ref[...]
    nb = nb_ref[...]
    q = jnp.dot(xv, wq_ref[...], preferred_element_type=_f32)
    k = jnp.dot(xv, wk_ref[...], preferred_element_type=_f32)
    v = jnp.dot(xv, wv_ref[...], preferred_element_type=_f32)
    va_ref[...] = v[:, :CH]
    vb_ref[...] = v[:, CH:]
    ad_ref[...] = jnp.dot(jnp.tanh(q + nb), w1_ref[...], preferred_element_type=_f32)
    as_ref[...] = jnp.dot(jnp.tanh(k + nb), w2_ref[...], preferred_element_type=_f32)


def _tc_edges_body(ea_ref, we_ref, w4_ref, rel_ref, w3_ref, out_ref):
    eb = jnp.dot(ea_ref[...], we_ref[...], preferred_element_type=_f32)
    a4 = jnp.dot(jnp.tanh(eb), w4_ref[...], preferred_element_type=_f32)
    crel = jnp.dot(jnp.tanh(rel_ref[...]), w3_ref[...], preferred_element_type=_f32)
    out_ref[...] = a4 + crel[0, 0]


def _tc_final_body(a0a_ref, a1a_ref, a0b_ref, a1b_ref, t0_ref, t1_ref,
                   den0_ref, den1_ref, x_ref, rel_ref, we_ref, wout_ref,
                   wroot_ref, b_ref, out_ref):
    d = den0_ref[...] + den1_ref[...]
    inv = 1.0 / (d + 1e-16)
    s = d * inv
    agg_a = (a0a_ref[...] + a1a_ref[...]) * inv
    agg_b = (a0b_ref[...] + a1b_ref[...]) * inv
    agg = (jnp.concatenate([agg_a, agg_b], axis=1)
           + s * rel_ref[...]
           + jnp.dot((t0_ref[...] + t1_ref[...]) * inv, we_ref[...],
                     preferred_element_type=_f32))
    out_ref[...] = (jnp.dot(agg, wout_ref[...], preferred_element_type=_f32)
                    + jnp.dot(x_ref[...], wroot_ref[...], preferred_element_type=_f32)
                    + b_ref[...])


@functools.cache
def _sc_scores_kernel():
    mesh = plsc.VectorSubcoreMesh(core_axis_name="c", subcore_axis_name="s")

    @functools.partial(
        pl.kernel,
        out_type=[jax.ShapeDtypeStruct((NW, NR, RW), _f32),
                  jax.ShapeDtypeStruct((N,), _f32),
                  jax.ShapeDtypeStruct((N,), _f32)],
        mesh=mesh,
        compiler_params=_SC_PARAMS,
        scratch_types=[
            pltpu.VMEM((N,), _f32),
            pltpu.VMEM((N,), _f32),
            pltpu.VMEM((NR, RW), _i32),
            pltpu.VMEM((NR, RW), _i32),
            pltpu.VMEM((NR, RW), _f32),
            pltpu.VMEM((ZR,), _f32),
            pltpu.VMEM_SHARED((NPAD,), _f32),
            pltpu.SemaphoreType.DMA,
        ],
    )
    def sc_scores(col_hbm, row_hbm, ea_hbm, ad_hbm, as_hbm,
                  ex_out, den0_out, den1_out,
                  adbuf, asbuf, colbuf, rowbuf, eabuf, zbuf, denom_sh, dsem):
        c = lax.axis_index("c")
        s = lax.axis_index("s")
        wid = s * NC + c

        z16 = jnp.zeros((16,), _f32)

        def zfill(i, carry):
            zbuf[pl.ds(i * 16, 16)] = z16
            return carry

        lax.fori_loop(0, ZR // 16, zfill, 0)
        pltpu.sync_copy(zbuf, denom_sh.at[pl.ds(pl.multiple_of(s * ZR, ZR), ZR)])

        pltpu.sync_copy(ad_hbm, adbuf)
        pltpu.sync_copy(as_hbm, asbuf)
        pltpu.sync_copy(col_hbm.at[wid], colbuf)
        pltpu.sync_copy(row_hbm.at[wid], rowbuf)
        pltpu.sync_copy(ea_hbm.at[wid], eabuf)
        plsc.subcore_barrier()

        def body(r, carry):
            def kbody(kk, kcarry):
                sl = pl.ds(kk * 16, 16)
                adv = plsc.load_gather(adbuf, [colbuf[r, sl]])
                asv = plsc.load_gather(asbuf, [rowbuf[r, sl]])
                eabuf[r, sl] = jnp.exp(adv + asv + eabuf[r, sl])
                return kcarry

            lax.fori_loop(0, RW // 16, kbody, 0)
            pltpu.async_copy(eabuf.at[r], denom_sh.at[colbuf.at[r]], dsem,
                             add=True)
            return carry

        lax.fori_loop(0, NR, body, 0)

        def ddrain(r, carry):
            pltpu.make_async_copy(eabuf.at[0], denom_sh.at[colbuf.at[0]],
                                  dsem).wait()
            return carry

        lax.fori_loop(0, NR, ddrain, 0)
        pltpu.sync_copy(eabuf, ex_out.at[wid])
        plsc.subcore_barrier()

        @pl.when((s == 0) & (c == 0))
        def _dump0():
            pltpu.sync_copy(denom_sh.at[pl.ds(0, N)], adbuf)
            pltpu.sync_copy(adbuf, den0_out)

        @pl.when((s == 0) & (c == 1))
        def _dump1():
            pltpu.sync_copy(denom_sh.at[pl.ds(0, N)], adbuf)
            pltpu.sync_copy(adbuf, den1_out)

    return sc_scores


@functools.cache
def _sc_aggregate_kernel():
    mesh = plsc.VectorSubcoreMesh(core_axis_name="c", subcore_axis_name="s")

    @functools.partial(
        pl.kernel,
        out_type=[jax.ShapeDtypeStruct((N, CH), _f32),
                  jax.ShapeDtypeStruct((N, CH), _f32),
                  jax.ShapeDtypeStruct((N, CH), _f32),
                  jax.ShapeDtypeStruct((N, CH), _f32),
                  jax.ShapeDtypeStruct((N, D_EDGE), _f32),
                  jax.ShapeDtypeStruct((N, D_EDGE), _f32)],
        mesh=mesh,
        compiler_params=_SC_PARAMS,
        scratch_types=[
            pltpu.VMEM((NR, RW), _i32),
            pltpu.VMEM((NR, RW), _i32),
            pltpu.VMEM((NR, RW), _f32),
            pltpu.VMEM((NB, RW, CH), _f32),
            pltpu.VMEM((NB, RW, D_EDGE), _f32),
            pltpu.VMEM((RW, CH), _f32),
            pltpu.VMEM((RW, D_EDGE), _f32),
            pltpu.VMEM_SHARED((N, CH), _f32),
            pltpu.VMEM_SHARED((N, D_EDGE), _f32),
            pltpu.SemaphoreType.DMA,
            pltpu.SemaphoreType.DMA,
            pltpu.SemaphoreType.DMA,
        ],
    )
    def sc_aggregate(col_hbm, row_hbm, ex_hbm, va_hbm, vb_hbm, eattr_hbm,
                     a0a_out, a1a_out, a0b_out, a1b_out, t0_out, t1_out,
                     colbuf, rowbuf, exbuf, vrows, earows,
                     zbufv, zbuf2, aggv_sh, t_sh, gsem, ssem, tsem):
        c = lax.axis_index("c")
        s = lax.axis_index("s")
        wid = s * NC + c
        ebase = wid * EW

        z16 = jnp.zeros((16,), _f32)

        def zfillv(i, carry):
            zbufv[i // (CH // 16), pl.ds((i % (CH // 16)) * 16, 16)] = z16
            return carry

        lax.fori_loop(0, RW * (CH // 16), zfillv, 0)

        def zfill2(i, carry):
            zbuf2[i, pl.ds(0, 16)] = z16
            return carry

        lax.fori_loop(0, RW, zfill2, 0)

        nchunk_z = jnp.where(s < NS - 1, ZR // RW, (N - (NS - 1) * ZR) // RW)

        def zero_aggv():
            def zc(j, carry):
                zoff = pl.multiple_of(s * ZR + j * RW, RW)
                pltpu.sync_copy(zbufv, aggv_sh.at[pl.ds(zoff, RW)])
                return carry

            lax.fori_loop(0, nchunk_z, zc, 0)

        zero_aggv()

        def ztc(j, carry):
            zoff = pl.multiple_of(s * ZR + j * RW, RW)
            pltpu.sync_copy(zbuf2, t_sh.at[pl.ds(zoff, RW)])
            return carry

        lax.fori_loop(0, nchunk_z, ztc, 0)

        pltpu.sync_copy(col_hbm.at[wid], colbuf)
        pltpu.sync_copy(row_hbm.at[wid], rowbuf)
        pltpu.sync_copy(ex_hbm.at[wid], exbuf)
        plsc.subcore_barrier()

        NWIN = NR // NB

        def edge_pass(v_hbm, first):
            def window(g, carry):
                ghs = []
                for b in range(NB):
                    ghs.append(pltpu.async_copy(
                        v_hbm.at[rowbuf.at[g * NB + b]], vrows.at[b], gsem))
                shs = []
                ths = []
                for b in range(NB):
                    r = g * NB + b
                    rv16 = jnp.full((16,), r, _i32)
                    ghs[b].wait()
                    if first:
                        eoff = pl.multiple_of(ebase + r * RW, RW)
                        pltpu.sync_copy(
                            eattr_hbm.at[pl.ds(eoff, RW)], earows.at[b])

                    def kbody(kk, kcarry, rv16=rv16, b=b):
                        for j in range(16):
                            e = kk * 16 + j
                            ev16 = jnp.full((16,), e, _i32)
                            bc = plsc.load_gather(exbuf, [rv16, ev16])
                            for f in range(CH // 16):
                                fs = pl.ds(f * 16, 16)
                                vrows[b, e, fs] = vrows[b, e, fs] * bc
                            if first:
                                earows[b, e, pl.ds(0, 16)] = (
                                    earows[b, e, pl.ds(0, 16)] * bc)
                        return kcarry

                    lax.fori_loop(0, RW // 16, kbody, 0)
                    shs.append(pltpu.async_copy(
                        vrows.at[b], aggv_sh.at[colbuf.at[r]], ssem,
                        add=True))
                    if first:
                        ths.append(pltpu.async_copy(
                            earows.at[b], t_sh.at[colbuf.at[r]], tsem,
                            add=True))
                for h in shs:
                    h.wait()
                for h in ths:
                    h.wait()
                return carry

            lax.fori_loop(0, NWIN, window, 0)

        def dump_aggv(out0, out1):
            def dump_chunk(j, out_ref):
                doff = pl.multiple_of(s * ZR + j * RW, RW)
                pltpu.sync_copy(aggv_sh.at[pl.ds(doff, RW)], vrows.at[0])
                pltpu.sync_copy(vrows.at[0], out_ref.at[pl.ds(doff, RW)])
                return 0

            @pl.when(c == 0)
            def _d0():
                lax.fori_loop(0, nchunk_z, lambda j, cy: dump_chunk(j, out0), 0)

            @pl.when(c == 1)
            def _d1():
                lax.fori_loop(0, nchunk_z, lambda j, cy: dump_chunk(j, out1), 0)

        edge_pass(va_hbm, True)
        plsc.subcore_barrier()
        dump_aggv(a0a_out, a1a_out)

        def dump_t(j, out_ref):
            doff = pl.multiple_of(s * ZR + j * RW, RW)
            pltpu.sync_copy(t_sh.at[pl.ds(doff, RW)], earows.at[0])
            pltpu.sync_copy(earows.at[0], out_ref.at[pl.ds(doff, RW)])
            return 0

        @pl.when(c == 0)
        def _dt0():
            lax.fori_loop(0, nchunk_z, lambda j, cy: dump_t(j, t0_out), 0)

        @pl.when(c == 1)
        def _dt1():
            lax.fori_loop(0, nchunk_z, lambda j, cy: dump_t(j, t1_out), 0)

        zero_aggv()
        plsc.subcore_barrier()

        edge_pass(vb_hbm, False)
        plsc.subcore_barrier()
        dump_aggv(a0b_out, a1b_out)

    return sc_aggregate


def kernel(x, edge_index, edge_attr, Wq, Wk, Wv, node_emb, rel_emb, W_edge,
           W_attn, W_out, b_out, W_root):
    row3 = edge_index[0].reshape(NW, NR, RW)
    col3 = edge_index[1].reshape(NW, NR, RW)
    w1 = W_attn[:C]
    w2 = W_attn[C:2 * C]
    w3 = W_attn[2 * C:3 * C]
    w4 = W_attn[3 * C:]
    nb = node_emb.reshape(1, C)
    rel = rel_emb.reshape(1, C)
    b2 = b_out.reshape(1, C)

    grid_n = 10
    bn = N // grid_n
    va, vb, ad, as_ = pl.pallas_call(
        _tc_nodes_body,
        grid=(grid_n,),
        in_specs=[
            pl.BlockSpec((bn, C), lambda i: (i, 0)),
            pl.BlockSpec((C, C), lambda i: (0, 0)),
            pl.BlockSpec((C, C), lambda i: (0, 0)),
            pl.BlockSpec((C, C), lambda i: (0, 0)),
            pl.BlockSpec((1, C), lambda i: (0, 0)),
            pl.BlockSpec((C, 1), lambda i: (0, 0)),
            pl.BlockSpec((C, 1), lambda i: (0, 0)),
        ],
        out_specs=[
            pl.BlockSpec((bn, CH), lambda i: (i, 0)),
            pl.BlockSpec((bn, CH), lambda i: (i, 0)),
            pl.BlockSpec((bn, 1), lambda i: (i, 0)),
            pl.BlockSpec((bn, 1), lambda i: (i, 0)),
        ],
        out_shape=[
            jax.ShapeDtypeStruct((N, CH), _f32),
            jax.ShapeDtypeStruct((N, CH), _f32),
            jax.ShapeDtypeStruct((N, 1), _f32),
            jax.ShapeDtypeStruct((N, 1), _f32),
        ],
    )(x, Wq, Wk, Wv, nb, w1, w2)

    grid_e = 100
    be = E // grid_e
    ea = pl.pallas_call(
        _tc_edges_body,
        grid=(grid_e,),
        in_specs=[
            pl.BlockSpec((be, D_EDGE), lambda i: (i, 0)),
            pl.BlockSpec((D_EDGE, C), lambda i: (0, 0)),
            pl.BlockSpec((C, 1), lambda i: (0, 0)),
            pl.BlockSpec((1, C), lambda i: (0, 0)),
            pl.BlockSpec((C, 1), lambda i: (0, 0)),
        ],
        out_specs=pl.BlockSpec((be, 1), lambda i: (i, 0)),
        out_shape=jax.ShapeDtypeStruct((E, 1), _f32),
    )(edge_attr, W_edge, w4, rel, w3)

    ex3, den0, den1 = _sc_scores_kernel()(col3, row3, ea.reshape(NW, NR, RW),
                                          ad.reshape(N), as_.reshape(N))

    a0a, a1a, a0b, a1b, t0, t1 = _sc_aggregate_kernel()(
        col3, row3, ex3, va, vb, edge_attr)

    out = pl.pallas_call(
        _tc_final_body,
        grid=(grid_n,),
        in_specs=[
            pl.BlockSpec((bn, CH), lambda i: (i, 0)),
            pl.BlockSpec((bn, CH), lambda i: (i, 0)),
            pl.BlockSpec((bn, CH), lambda i: (i, 0)),
            pl.BlockSpec((bn, CH), lambda i: (i, 0)),
            pl.BlockSpec((bn, D_EDGE), lambda i: (i, 0)),
            pl.BlockSpec((bn, D_EDGE), lambda i: (i, 0)),
            pl.BlockSpec((bn, 1), lambda i: (i, 0)),
            pl.BlockSpec((bn, 1), lambda i: (i, 0)),
            pl.BlockSpec((bn, C), lambda i: (i, 0)),
            pl.BlockSpec((1, C), lambda i: (0, 0)),
            pl.BlockSpec((D_EDGE, C), lambda i: (0, 0)),
            pl.BlockSpec((C, C), lambda i: (0, 0)),
            pl.BlockSpec((C, C), lambda i: (0, 0)),
            pl.BlockSpec((1, C), lambda i: (0, 0)),
        ],
        out_specs=pl.BlockSpec((bn, C), lambda i: (i, 0)),
        out_shape=jax.ShapeDtypeStruct((N, C), _f32),
    )(a0a, a1a, a0b, a1b, t0, t1, den0.reshape(N, 1), den1.reshape(N, 1),
      x, rel, W_edge, W_out, W_root, b2)

    return out


# speedup vs baseline: 1.0020x; 1.0005x over previous
import functools

import jax
import jax.numpy as jnp
from jax import lax
from jax.experimental import pallas as pl
from jax.experimental.pallas import tpu as pltpu
from jax.experimental.pallas import tpu_sc as plsc

N = 10000
E = 320000
C = 128
CH = C // 2
D_EDGE = 16

NC = 2
NS = 16
NW = NC * NS
EW = E // NW
RW = 80
NR = EW // RW
NPAD = 10240
ZR = NPAD // NS
NB = 5

_f32 = jnp.float32
_i32 = jnp.int32

_SC_PARAMS = pltpu.CompilerParams(
    needs_layout_passes=False, use_tc_tiling_on_sc=False)


def _tc_nodes_body(x_ref, wq_ref, wk_ref, wv_ref, nb_ref, w1_ref, w2_ref,
                   va_ref, vb_ref, ad_ref, as_ref):
    xv = x_ref[...]
    nb = nb_ref[...]
    q = jnp.dot(xv, wq_ref[...], preferred_element_type=_f32)
    k = jnp.dot(xv, wk_ref[...], preferred_element_type=_f32)
    v = jnp.dot(xv, wv_ref[...], preferred_element_type=_f32)
    va_ref[...] = v[:, :CH]
    vb_ref[...] = v[:, CH:]
    ad_ref[...] = jnp.dot(jnp.tanh(q + nb), w1_ref[...], preferred_element_type=_f32)
    as_ref[...] = jnp.dot(jnp.tanh(k + nb), w2_ref[...], preferred_element_type=_f32)


def _tc_edges_body(ea_ref, we_ref, w4_ref, rel_ref, w3_ref, out_ref):
    eb = jnp.dot(ea_ref[...], we_ref[...], preferred_element_type=_f32)
    a4 = jnp.dot(jnp.tanh(eb), w4_ref[...], preferred_element_type=_f32)
    crel = jnp.dot(jnp.tanh(rel_ref[...]), w3_ref[...], preferred_element_type=_f32)
    out_ref[...] = a4 + crel[0, 0]


def _tc_final_body(a0a_ref, a1a_ref, a0b_ref, a1b_ref, t0_ref, t1_ref,
                   den0_ref, den1_ref, x_ref, rel_ref, we_ref, wout_ref,
                   wroot_ref, b_ref, out_ref):
    d = den0_ref[...] + den1_ref[...]
    inv = 1.0 / (d + 1e-16)
    s = d * inv
    agg_a = (a0a_ref[...] + a1a_ref[...]) * inv
    agg_b = (a0b_ref[...] + a1b_ref[...]) * inv
    agg = (jnp.concatenate([agg_a, agg_b], axis=1)
           + s * rel_ref[...]
           + jnp.dot((t0_ref[...] + t1_ref[...]) * inv, we_ref[...],
                     preferred_element_type=_f32))
    out_ref[...] = (jnp.dot(agg, wout_ref[...], preferred_element_type=_f32)
                    + jnp.dot(x_ref[...], wroot_ref[...], preferred_element_type=_f32)
                    + b_ref[...])


@functools.cache
def _sc_scores_kernel():
    mesh = plsc.VectorSubcoreMesh(core_axis_name="c", subcore_axis_name="s")

    @functools.partial(
        pl.kernel,
        out_type=[jax.ShapeDtypeStruct((NW, NR, RW), _f32),
                  jax.ShapeDtypeStruct((N,), _f32),
                  jax.ShapeDtypeStruct((N,), _f32)],
        mesh=mesh,
        compiler_params=_SC_PARAMS,
        scratch_types=[
            pltpu.VMEM((N,), _f32),
            pltpu.VMEM((N,), _f32),
            pltpu.VMEM((NR, RW), _i32),
            pltpu.VMEM((NR, RW), _i32),
            pltpu.VMEM((NR, RW), _f32),
            pltpu.VMEM((ZR,), _f32),
            pltpu.VMEM_SHARED((NPAD,), _f32),
            pltpu.SemaphoreType.DMA,
        ],
    )
    def sc_scores(col_hbm, row_hbm, ea_hbm, ad_hbm, as_hbm,
                  ex_out, den0_out, den1_out,
                  adbuf, asbuf, colbuf, rowbuf, eabuf, zbuf, denom_sh, dsem):
        c = lax.axis_index("c")
        s = lax.axis_index("s")
        wid = s * NC + c

        z16 = jnp.zeros((16,), _f32)

        def zfill(i, carry):
            zbuf[pl.ds(i * 16, 16)] = z16
            return carry

        lax.fori_loop(0, ZR // 16, zfill, 0)
        pltpu.sync_copy(zbuf, denom_sh.at[pl.ds(pl.multiple_of(s * ZR, ZR), ZR)])

        pltpu.sync_copy(ad_hbm, adbuf)
        pltpu.sync_copy(as_hbm, asbuf)
        pltpu.sync_copy(col_hbm.at[wid], colbuf)
        pltpu.sync_copy(row_hbm.at[wid], rowbuf)
        pltpu.sync_copy(ea_hbm.at[wid], eabuf)
        plsc.subcore_barrier()

        def body(r, carry):
            def kbody(kk, kcarry):
                sl = pl.ds(kk * 16, 16)
                adv = plsc.load_gather(adbuf, [colbuf[r, sl]])
                asv = plsc.load_gather(asbuf, [rowbuf[r, sl]])
                eabuf[r, sl] = jnp.exp(adv + asv + eabuf[r, sl])
                return kcarry

            lax.fori_loop(0, RW // 16, kbody, 0)
            pltpu.sync_copy(eabuf.at[r], denom_sh.at[colbuf.at[r]], add=True)
            return carry

        lax.fori_loop(0, NR, body, 0)
        pltpu.sync_copy(eabuf, ex_out.at[wid])
        plsc.subcore_barrier()

        @pl.when((s == 0) & (c == 0))
        def _dump0():
            pltpu.sync_copy(denom_sh.at[pl.ds(0, N)], adbuf)
            pltpu.sync_copy(adbuf, den0_out)

        @pl.when((s == 0) & (c == 1))
        def _dump1():
            pltpu.sync_copy(denom_sh.at[pl.ds(0, N)], adbuf)
            pltpu.sync_copy(adbuf, den1_out)

    return sc_scores


@functools.cache
def _sc_aggregate_kernel():
    mesh = plsc.VectorSubcoreMesh(core_axis_name="c", subcore_axis_name="s")

    @functools.partial(
        pl.kernel,
        out_type=[jax.ShapeDtypeStruct((N, CH), _f32),
                  jax.ShapeDtypeStruct((N, CH), _f32),
                  jax.ShapeDtypeStruct((N, CH), _f32),
                  jax.ShapeDtypeStruct((N, CH), _f32),
                  jax.ShapeDtypeStruct((N, D_EDGE), _f32),
                  jax.ShapeDtypeStruct((N, D_EDGE), _f32)],
        mesh=mesh,
        compiler_params=_SC_PARAMS,
        scratch_types=[
            pltpu.VMEM((NR, RW), _i32),
            pltpu.VMEM((NR, RW), _i32),
            pltpu.VMEM((NR, RW), _f32),
            pltpu.VMEM((NB, RW, CH), _f32),
            pltpu.VMEM((NB, RW, D_EDGE), _f32),
            pltpu.VMEM((RW, CH), _f32),
            pltpu.VMEM((RW, D_EDGE), _f32),
            pltpu.VMEM_SHARED((N, CH), _f32),
            pltpu.VMEM_SHARED((N, D_EDGE), _f32),
            pltpu.SemaphoreType.DMA,
            pltpu.SemaphoreType.DMA,
            pltpu.SemaphoreType.DMA,
        ],
    )
    def sc_aggregate(col_hbm, row_hbm, ex_hbm, va_hbm, vb_hbm, eattr_hbm,
                     a0a_out, a1a_out, a0b_out, a1b_out, t0_out, t1_out,
                     colbuf, rowbuf, exbuf, vrows, earows,
                     zbufv, zbuf2, aggv_sh, t_sh, gsem, ssem, tsem):
        c = lax.axis_index("c")
        s = lax.axis_index("s")
        wid = s * NC + c
        ebase = wid * EW

        z16 = jnp.zeros((16,), _f32)

        def zfillv(i, carry):
            zbufv[i // (CH // 16), pl.ds((i % (CH // 16)) * 16, 16)] = z16
            return carry

        lax.fori_loop(0, RW * (CH // 16), zfillv, 0)

        def zfill2(i, carry):
            zbuf2[i, pl.ds(0, 16)] = z16
            return carry

        lax.fori_loop(0, RW, zfill2, 0)

        nchunk_z = jnp.where(s < NS - 1, ZR // RW, (N - (NS - 1) * ZR) // RW)

        def zero_aggv():
            def zc(j, carry):
                zoff = pl.multiple_of(s * ZR + j * RW, RW)
                pltpu.sync_copy(zbufv, aggv_sh.at[pl.ds(zoff, RW)])
                return carry

            lax.fori_loop(0, nchunk_z, zc, 0)

        zero_aggv()

        def ztc(j, carry):
            zoff = pl.multiple_of(s * ZR + j * RW, RW)
            pltpu.sync_copy(zbuf2, t_sh.at[pl.ds(zoff, RW)])
            return carry

        lax.fori_loop(0, nchunk_z, ztc, 0)

        pltpu.sync_copy(col_hbm.at[wid], colbuf)
        pltpu.sync_copy(row_hbm.at[wid], rowbuf)
        pltpu.sync_copy(ex_hbm.at[wid], exbuf)
        plsc.subcore_barrier()

        NWIN = NR // NB

        def edge_pass(v_hbm, first):
            def window(g, carry):
                ghs = []
                for b in range(NB):
                    ghs.append(pltpu.async_copy(
                        v_hbm.at[rowbuf.at[g * NB + b]], vrows.at[b], gsem))
                shs = []
                ths = []
                for b in range(NB):
                    r = g * NB + b
                    rv16 = jnp.full((16,), r, _i32)
                    ghs[b].wait()
                    if first:
                        eoff = pl.multiple_of(ebase + r * RW, RW)
                        pltpu.sync_copy(
                            eattr_hbm.at[pl.ds(eoff, RW)], earows.at[b])

                    def kbody(kk, kcarry, rv16=rv16, b=b):
                        for j in range(16):
                            e = kk * 16 + j
                            ev16 = jnp.full((16,), e, _i32)
                            bc = plsc.load_gather(exbuf, [rv16, ev16])
                            for f in range(CH // 16):
                                fs = pl.ds(f * 16, 16)
                                vrows[b, e, fs] = vrows[b, e, fs] * bc
                            if first:
                                earows[b, e, pl.ds(0, 16)] = (
                                    earows[b, e, pl.ds(0, 16)] * bc)
                        return kcarry

                    lax.fori_loop(0, RW // 16, kbody, 0)
                    shs.append(pltpu.async_copy(
                        vrows.at[b], aggv_sh.at[colbuf.at[r]], ssem,
                        add=True))
                    if first:
                        ths.append(pltpu.async_copy(
                            earows.at[b], t_sh.at[colbuf.at[r]], tsem,
                            add=True))
                for h in shs:
                    h.wait()
                for h in ths:
                    h.wait()
                return carry

            lax.fori_loop(0, NWIN, window, 0)

        def dump_aggv(out0, out1):
            def dump_chunk(j, out_ref):
                doff = pl.multiple_of(s * ZR + j * RW, RW)
                pltpu.sync_copy(aggv_sh.at[pl.ds(doff, RW)], vrows.at[0])
                pltpu.sync_copy(vrows.at[0], out_ref.at[pl.ds(doff, RW)])
                return 0

            @pl.when(c == 0)
            def _d0():
                lax.fori_loop(0, nchunk_z, lambda j, cy: dump_chunk(j, out0), 0)

            @pl.when(c == 1)
            def _d1():
                lax.fori_loop(0, nchunk_z, lambda j, cy: dump_chunk(j, out1), 0)

        edge_pass(va_hbm, True)
        plsc.subcore_barrier()
        dump_aggv(a0a_out, a1a_out)

        def dump_t(j, out_ref):
            doff = pl.multiple_of(s * ZR + j * RW, RW)
            pltpu.sync_copy(t_sh.at[pl.ds(doff, RW)], earows.at[0])
            pltpu.sync_copy(earows.at[0], out_ref.at[pl.ds(doff, RW)])
            return 0

        @pl.when(c == 0)
        def _dt0():
            lax.fori_loop(0, nchunk_z, lambda j, cy: dump_t(j, t0_out), 0)

        @pl.when(c == 1)
        def _dt1():
            lax.fori_loop(0, nchunk_z, lambda j, cy: dump_t(j, t1_out), 0)

        zero_aggv()
        plsc.subcore_barrier()

        edge_pass(vb_hbm, False)
        plsc.subcore_barrier()
        dump_aggv(a0b_out, a1b_out)

    return sc_aggregate


def kernel(x, edge_index, edge_attr, Wq, Wk, Wv, node_emb, rel_emb, W_edge,
           W_attn, W_out, b_out, W_root):
    row3 = edge_index[0].reshape(NW, NR, RW)
    col3 = edge_index[1].reshape(NW, NR, RW)
    w1 = W_attn[:C]
    w2 = W_attn[C:2 * C]
    w3 = W_attn[2 * C:3 * C]
    w4 = W_attn[3 * C:]
    nb = node_emb.reshape(1, C)
    rel = rel_emb.reshape(1, C)
    b2 = b_out.reshape(1, C)

    grid_n = 10
    bn = N // grid_n
    va, vb, ad, as_ = pl.pallas_call(
        _tc_nodes_body,
        grid=(grid_n,),
        in_specs=[
            pl.BlockSpec((bn, C), lambda i: (i, 0)),
            pl.BlockSpec((C, C), lambda i: (0, 0)),
            pl.BlockSpec((C, C), lambda i: (0, 0)),
            pl.BlockSpec((C, C), lambda i: (0, 0)),
            pl.BlockSpec((1, C), lambda i: (0, 0)),
            pl.BlockSpec((C, 1), lambda i: (0, 0)),
            pl.BlockSpec((C, 1), lambda i: (0, 0)),
        ],
        out_specs=[
            pl.BlockSpec((bn, CH), lambda i: (i, 0)),
            pl.BlockSpec((bn, CH), lambda i: (i, 0)),
            pl.BlockSpec((bn, 1), lambda i: (i, 0)),
            pl.BlockSpec((bn, 1), lambda i: (i, 0)),
        ],
        out_shape=[
            jax.ShapeDtypeStruct((N, CH), _f32),
            jax.ShapeDtypeStruct((N, CH), _f32),
            jax.ShapeDtypeStruct((N, 1), _f32),
            jax.ShapeDtypeStruct((N, 1), _f32),
        ],
    )(x, Wq, Wk, Wv, nb, w1, w2)

    grid_e = 100
    be = E // grid_e
    ea = pl.pallas_call(
        _tc_edges_body,
        grid=(grid_e,),
        in_specs=[
            pl.BlockSpec((be, D_EDGE), lambda i: (i, 0)),
            pl.BlockSpec((D_EDGE, C), lambda i: (0, 0)),
            pl.BlockSpec((C, 1), lambda i: (0, 0)),
            pl.BlockSpec((1, C), lambda i: (0, 0)),
            pl.BlockSpec((C, 1), lambda i: (0, 0)),
        ],
        out_specs=pl.BlockSpec((be, 1), lambda i: (i, 0)),
        out_shape=jax.ShapeDtypeStruct((E, 1), _f32),
    )(edge_attr, W_edge, w4, rel, w3)

    ex3, den0, den1 = _sc_scores_kernel()(col3, row3, ea.reshape(NW, NR, RW),
                                          ad.reshape(N), as_.reshape(N))

    a0a, a1a, a0b, a1b, t0, t1 = _sc_aggregate_kernel()(
        col3, row3, ex3, va, vb, edge_attr)

    out = pl.pallas_call(
        _tc_final_body,
        grid=(grid_n,),
        in_specs=[
            pl.BlockSpec((bn, CH), lambda i: (i, 0)),
            pl.BlockSpec((bn, CH), lambda i: (i, 0)),
            pl.BlockSpec((bn, CH), lambda i: (i, 0)),
            pl.BlockSpec((bn, CH), lambda i: (i, 0)),
            pl.BlockSpec((bn, D_EDGE), lambda i: (i, 0)),
            pl.BlockSpec((bn, D_EDGE), lambda i: (i, 0)),
            pl.BlockSpec((bn, 1), lambda i: (i, 0)),
            pl.BlockSpec((bn, 1), lambda i: (i, 0)),
            pl.BlockSpec((bn, C), lambda i: (i, 0)),
            pl.BlockSpec((1, C), lambda i: (0, 0)),
            pl.BlockSpec((D_EDGE, C), lambda i: (0, 0)),
            pl.BlockSpec((C, C), lambda i: (0, 0)),
            pl.BlockSpec((C, C), lambda i: (0, 0)),
            pl.BlockSpec((1, C), lambda i: (0, 0)),
        ],
        out_specs=pl.BlockSpec((bn, C), lambda i: (i, 0)),
        out_shape=jax.ShapeDtypeStruct((N, C), _f32),
    )(a0a, a1a, a0b, a1b, t0, t1, den0.reshape(N, 1), den1.reshape(N, 1),
      x, rel, W_edge, W_out, W_root, b2)

    return out


# speedup vs baseline: 1.0738x; 1.0716x over previous
import functools

import jax
import jax.numpy as jnp
from jax import lax
from jax.experimental import pallas as pl
from jax.experimental.pallas import tpu as pltpu
from jax.experimental.pallas import tpu_sc as plsc

N = 10000
E = 320000
C = 128
CH = C // 2
D_EDGE = 16

NC = 2
NS = 16
NW = NC * NS
EW = E // NW
RW = 80
NR = EW // RW
NPAD = 10240
ZR = NPAD // NS
NB = 5

_f32 = jnp.float32
_i32 = jnp.int32

_SC_PARAMS = pltpu.CompilerParams(
    needs_layout_passes=False, use_tc_tiling_on_sc=False)


def _tc_nodes_body(x_ref, wq_ref, wk_ref, wv_ref, nb_ref, w1_ref, w2_ref,
                   va_ref, vb_ref, ad_ref, as_ref):
    xv = x_ref[...]
    nb = nb_ref[...]
    q = jnp.dot(xv, wq_ref[...], preferred_element_type=_f32)
    k = jnp.dot(xv, wk_ref[...], preferred_element_type=_f32)
    v = jnp.dot(xv, wv_ref[...], preferred_element_type=_f32)
    va_ref[...] = v[:, :CH]
    vb_ref[...] = v[:, CH:]
    ad_ref[...] = jnp.dot(jnp.tanh(q + nb), w1_ref[...], preferred_element_type=_f32)
    as_ref[...] = jnp.dot(jnp.tanh(k + nb), w2_ref[...], preferred_element_type=_f32)


def _tc_edges_body(ea_ref, we_ref, w4_ref, rel_ref, w3_ref, out_ref):
    eb = jnp.dot(ea_ref[...], we_ref[...], preferred_element_type=_f32)
    a4 = jnp.dot(jnp.tanh(eb), w4_ref[...], preferred_element_type=_f32)
    crel = jnp.dot(jnp.tanh(rel_ref[...]), w3_ref[...], preferred_element_type=_f32)
    out_ref[...] = a4 + crel[0, 0]


def _tc_final_body(a0a_ref, a1a_ref, a0b_ref, a1b_ref, t0_ref, t1_ref,
                   den0_ref, den1_ref, x_ref, rel_ref, we_ref, wout_ref,
                   wroot_ref, b_ref, out_ref):
    d = den0_ref[...] + den1_ref[...]
    inv = 1.0 / (d + 1e-16)
    s = d * inv
    agg_a = (a0a_ref[...] + a1a_ref[...]) * inv
    agg_b = (a0b_ref[...] + a1b_ref[...]) * inv
    agg = (jnp.concatenate([agg_a, agg_b], axis=1)
           + s * rel_ref[...]
           + jnp.dot((t0_ref[...] + t1_ref[...]) * inv, we_ref[...],
                     preferred_element_type=_f32))
    out_ref[...] = (jnp.dot(agg, wout_ref[...], preferred_element_type=_f32)
                    + jnp.dot(x_ref[...], wroot_ref[...], preferred_element_type=_f32)
                    + b_ref[...])


@functools.cache
def _sc_scores_kernel():
    mesh = plsc.VectorSubcoreMesh(core_axis_name="c", subcore_axis_name="s")

    @functools.partial(
        pl.kernel,
        out_type=[jax.ShapeDtypeStruct((NW, NR, RW), _f32),
                  jax.ShapeDtypeStruct((N,), _f32),
                  jax.ShapeDtypeStruct((N,), _f32)],
        mesh=mesh,
        compiler_params=_SC_PARAMS,
        scratch_types=[
            pltpu.VMEM((N,), _f32),
            pltpu.VMEM((N,), _f32),
            pltpu.VMEM((NR, RW), _i32),
            pltpu.VMEM((NR, RW), _i32),
            pltpu.VMEM((NR, RW), _f32),
            pltpu.VMEM((ZR,), _f32),
            pltpu.VMEM_SHARED((NPAD,), _f32),
            pltpu.SemaphoreType.DMA,
        ],
    )
    def sc_scores(col_hbm, row_hbm, ea_hbm, ad_hbm, as_hbm,
                  ex_out, den0_out, den1_out,
                  adbuf, asbuf, colbuf, rowbuf, eabuf, zbuf, denom_sh, dsem):
        c = lax.axis_index("c")
        s = lax.axis_index("s")
        wid = s * NC + c

        z16 = jnp.zeros((16,), _f32)

        def zfill(i, carry):
            zbuf[pl.ds(i * 16, 16)] = z16
            return carry

        lax.fori_loop(0, ZR // 16, zfill, 0)
        pltpu.sync_copy(zbuf, denom_sh.at[pl.ds(pl.multiple_of(s * ZR, ZR), ZR)])

        pltpu.sync_copy(ad_hbm, adbuf)
        pltpu.sync_copy(as_hbm, asbuf)
        pltpu.sync_copy(col_hbm.at[wid], colbuf)
        pltpu.sync_copy(row_hbm.at[wid], rowbuf)
        pltpu.sync_copy(ea_hbm.at[wid], eabuf)
        plsc.subcore_barrier()

        def body(r, carry):
            def kbody(kk, kcarry):
                sl = pl.ds(kk * 16, 16)
                adv = plsc.load_gather(adbuf, [colbuf[r, sl]])
                asv = plsc.load_gather(asbuf, [rowbuf[r, sl]])
                eabuf[r, sl] = jnp.exp(adv + asv + eabuf[r, sl])
                return kcarry

            lax.fori_loop(0, RW // 16, kbody, 0)
            pltpu.sync_copy(eabuf.at[r], denom_sh.at[colbuf.at[r]], add=True)
            return carry

        lax.fori_loop(0, NR, body, 0)
        pltpu.sync_copy(eabuf, ex_out.at[wid])
        plsc.subcore_barrier()

        @pl.when((s == 0) & (c == 0))
        def _dump0():
            pltpu.sync_copy(denom_sh.at[pl.ds(0, N)], adbuf)
            pltpu.sync_copy(adbuf, den0_out)

        @pl.when((s == 0) & (c == 1))
        def _dump1():
            pltpu.sync_copy(denom_sh.at[pl.ds(0, N)], adbuf)
            pltpu.sync_copy(adbuf, den1_out)

    return sc_scores


@functools.cache
def _sc_aggregate_kernel():
    mesh = plsc.VectorSubcoreMesh(core_axis_name="c", subcore_axis_name="s")

    @functools.partial(
        pl.kernel,
        out_type=[jax.ShapeDtypeStruct((N, CH), _f32),
                  jax.ShapeDtypeStruct((N, CH), _f32),
                  jax.ShapeDtypeStruct((N, CH), _f32),
                  jax.ShapeDtypeStruct((N, CH), _f32),
                  jax.ShapeDtypeStruct((N, D_EDGE), _f32),
                  jax.ShapeDtypeStruct((N, D_EDGE), _f32)],
        mesh=mesh,
        compiler_params=_SC_PARAMS,
        scratch_types=[
            pltpu.VMEM((NR, RW), _i32),
            pltpu.VMEM((NR, RW), _i32),
            pltpu.VMEM((NR, RW), _f32),
            pltpu.VMEM((NB, RW, CH), _f32),
            pltpu.VMEM((NB, RW, D_EDGE), _f32),
            pltpu.VMEM((RW, CH), _f32),
            pltpu.VMEM((RW, D_EDGE), _f32),
            pltpu.VMEM_SHARED((N, CH), _f32),
            pltpu.VMEM_SHARED((N, D_EDGE), _f32),
            pltpu.SemaphoreType.DMA,
            pltpu.SemaphoreType.DMA,
            pltpu.SemaphoreType.DMA,
            pltpu.SemaphoreType.DMA,
        ],
    )
    def sc_aggregate(col_hbm, row_hbm, ex_hbm, va_hbm, vb_hbm, eattr_hbm,
                     a0a_out, a1a_out, a0b_out, a1b_out, t0_out, t1_out,
                     colbuf, rowbuf, exbuf, vrows, earows,
                     zbufv, zbuf2, aggv_sh, t_sh, gsem, esem, ssem, tsem):
        c = lax.axis_index("c")
        s = lax.axis_index("s")
        wid = s * NC + c
        ebase = wid * EW

        z16 = jnp.zeros((16,), _f32)

        def zfillv(i, carry):
            zbufv[i // (CH // 16), pl.ds((i % (CH // 16)) * 16, 16)] = z16
            return carry

        lax.fori_loop(0, RW * (CH // 16), zfillv, 0)

        def zfill2(i, carry):
            zbuf2[i, pl.ds(0, 16)] = z16
            return carry

        lax.fori_loop(0, RW, zfill2, 0)

        nchunk_z = jnp.where(s < NS - 1, ZR // RW, (N - (NS - 1) * ZR) // RW)

        def zero_aggv():
            def zc(j, carry):
                zoff = pl.multiple_of(s * ZR + j * RW, RW)
                pltpu.sync_copy(zbufv, aggv_sh.at[pl.ds(zoff, RW)])
                return carry

            lax.fori_loop(0, nchunk_z, zc, 0)

        zero_aggv()

        def ztc(j, carry):
            zoff = pl.multiple_of(s * ZR + j * RW, RW)
            pltpu.sync_copy(zbuf2, t_sh.at[pl.ds(zoff, RW)])
            return carry

        lax.fori_loop(0, nchunk_z, ztc, 0)

        pltpu.sync_copy(col_hbm.at[wid], colbuf)
        pltpu.sync_copy(row_hbm.at[wid], rowbuf)
        pltpu.sync_copy(ex_hbm.at[wid], exbuf)
        plsc.subcore_barrier()

        NWIN = NR // NB

        def edge_pass(v_hbm, first):
            def window(g, carry):
                ghs = []
                ehs = []
                for b in range(NB):
                    r = g * NB + b
                    ghs.append(pltpu.async_copy(
                        v_hbm.at[rowbuf.at[r]], vrows.at[b], gsem))
                    if first:
                        eoff = pl.multiple_of(ebase + r * RW, RW)
                        ehs.append(pltpu.async_copy(
                            eattr_hbm.at[pl.ds(eoff, RW)], earows.at[b], esem))
                shs = []
                ths = []
                for b in range(NB):
                    r = g * NB + b
                    rv16 = jnp.full((16,), r, _i32)
                    ghs[b].wait()
                    if first:
                        ehs[b].wait()

                    def kbody(kk, kcarry, rv16=rv16, b=b):
                        for j in range(16):
                            e = kk * 16 + j
                            ev16 = jnp.full((16,), e, _i32)
                            bc = plsc.load_gather(exbuf, [rv16, ev16])
                            for f in range(CH // 16):
                                fs = pl.ds(f * 16, 16)
                                vrows[b, e, fs] = vrows[b, e, fs] * bc
                            if first:
                                earows[b, e, pl.ds(0, 16)] = (
                                    earows[b, e, pl.ds(0, 16)] * bc)
                        return kcarry

                    lax.fori_loop(0, RW // 16, kbody, 0)
                    shs.append(pltpu.async_copy(
                        vrows.at[b], aggv_sh.at[colbuf.at[r]], ssem,
                        add=True))
                    if first:
                        ths.append(pltpu.async_copy(
                            earows.at[b], t_sh.at[colbuf.at[r]], tsem,
                            add=True))
                for h in shs:
                    h.wait()
                for h in ths:
                    h.wait()
                return carry

            lax.fori_loop(0, NWIN, window, 0)

        def dump_aggv(out0, out1):
            def dump_chunk(j, out_ref):
                doff = pl.multiple_of(s * ZR + j * RW, RW)
                pltpu.sync_copy(aggv_sh.at[pl.ds(doff, RW)], vrows.at[0])
                pltpu.sync_copy(vrows.at[0], out_ref.at[pl.ds(doff, RW)])
                return 0

            @pl.when(c == 0)
            def _d0():
                lax.fori_loop(0, nchunk_z, lambda j, cy: dump_chunk(j, out0), 0)

            @pl.when(c == 1)
            def _d1():
                lax.fori_loop(0, nchunk_z, lambda j, cy: dump_chunk(j, out1), 0)

        edge_pass(va_hbm, True)
        plsc.subcore_barrier()
        dump_aggv(a0a_out, a1a_out)

        def dump_t(j, out_ref):
            doff = pl.multiple_of(s * ZR + j * RW, RW)
            pltpu.sync_copy(t_sh.at[pl.ds(doff, RW)], earows.at[0])
            pltpu.sync_copy(earows.at[0], out_ref.at[pl.ds(doff, RW)])
            return 0

        @pl.when(c == 0)
        def _dt0():
            lax.fori_loop(0, nchunk_z, lambda j, cy: dump_t(j, t0_out), 0)

        @pl.when(c == 1)
        def _dt1():
            lax.fori_loop(0, nchunk_z, lambda j, cy: dump_t(j, t1_out), 0)

        zero_aggv()
        plsc.subcore_barrier()

        edge_pass(vb_hbm, False)
        plsc.subcore_barrier()
        dump_aggv(a0b_out, a1b_out)

    return sc_aggregate


def kernel(x, edge_index, edge_attr, Wq, Wk, Wv, node_emb, rel_emb, W_edge,
           W_attn, W_out, b_out, W_root):
    row3 = edge_index[0].reshape(NW, NR, RW)
    col3 = edge_index[1].reshape(NW, NR, RW)
    w1 = W_attn[:C]
    w2 = W_attn[C:2 * C]
    w3 = W_attn[2 * C:3 * C]
    w4 = W_attn[3 * C:]
    nb = node_emb.reshape(1, C)
    rel = rel_emb.reshape(1, C)
    b2 = b_out.reshape(1, C)

    grid_n = 10
    bn = N // grid_n
    va, vb, ad, as_ = pl.pallas_call(
        _tc_nodes_body,
        grid=(grid_n,),
        in_specs=[
            pl.BlockSpec((bn, C), lambda i: (i, 0)),
            pl.BlockSpec((C, C), lambda i: (0, 0)),
            pl.BlockSpec((C, C), lambda i: (0, 0)),
            pl.BlockSpec((C, C), lambda i: (0, 0)),
            pl.BlockSpec((1, C), lambda i: (0, 0)),
            pl.BlockSpec((C, 1), lambda i: (0, 0)),
            pl.BlockSpec((C, 1), lambda i: (0, 0)),
        ],
        out_specs=[
            pl.BlockSpec((bn, CH), lambda i: (i, 0)),
            pl.BlockSpec((bn, CH), lambda i: (i, 0)),
            pl.BlockSpec((bn, 1), lambda i: (i, 0)),
            pl.BlockSpec((bn, 1), lambda i: (i, 0)),
        ],
        out_shape=[
            jax.ShapeDtypeStruct((N, CH), _f32),
            jax.ShapeDtypeStruct((N, CH), _f32),
            jax.ShapeDtypeStruct((N, 1), _f32),
            jax.ShapeDtypeStruct((N, 1), _f32),
        ],
    )(x, Wq, Wk, Wv, nb, w1, w2)

    grid_e = 100
    be = E // grid_e
    ea = pl.pallas_call(
        _tc_edges_body,
        grid=(grid_e,),
        in_specs=[
            pl.BlockSpec((be, D_EDGE), lambda i: (i, 0)),
            pl.BlockSpec((D_EDGE, C), lambda i: (0, 0)),
            pl.BlockSpec((C, 1), lambda i: (0, 0)),
            pl.BlockSpec((1, C), lambda i: (0, 0)),
            pl.BlockSpec((C, 1), lambda i: (0, 0)),
        ],
        out_specs=pl.BlockSpec((be, 1), lambda i: (i, 0)),
        out_shape=jax.ShapeDtypeStruct((E, 1), _f32),
    )(edge_attr, W_edge, w4, rel, w3)

    ex3, den0, den1 = _sc_scores_kernel()(col3, row3, ea.reshape(NW, NR, RW),
                                          ad.reshape(N), as_.reshape(N))

    a0a, a1a, a0b, a1b, t0, t1 = _sc_aggregate_kernel()(
        col3, row3, ex3, va, vb, edge_attr)

    out = pl.pallas_call(
        _tc_final_body,
        grid=(grid_n,),
        in_specs=[
            pl.BlockSpec((bn, CH), lambda i: (i, 0)),
            pl.BlockSpec((bn, CH), lambda i: (i, 0)),
            pl.BlockSpec((bn, CH), lambda i: (i, 0)),
            pl.BlockSpec((bn, CH), lambda i: (i, 0)),
            pl.BlockSpec((bn, D_EDGE), lambda i: (i, 0)),
            pl.BlockSpec((bn, D_EDGE), lambda i: (i, 0)),
            pl.BlockSpec((bn, 1), lambda i: (i, 0)),
            pl.BlockSpec((bn, 1), lambda i: (i, 0)),
            pl.BlockSpec((bn, C), lambda i: (i, 0)),
            pl.BlockSpec((1, C), lambda i: (0, 0)),
            pl.BlockSpec((D_EDGE, C), lambda i: (0, 0)),
            pl.BlockSpec((C, C), lambda i: (0, 0)),
            pl.BlockSpec((C, C), lambda i: (0, 0)),
            pl.BlockSpec((1, C), lambda i: (0, 0)),
        ],
        out_specs=pl.BlockSpec((bn, C), lambda i: (i, 0)),
        out_shape=jax.ShapeDtypeStruct((N, C), _f32),
    )(a0a, a1a, a0b, a1b, t0, t1, den0.reshape(N, 1), den1.reshape(N, 1),
      x, rel, W_edge, W_out, W_root, b2)

    return out
